# baseline (speedup 1.0000x reference)
import functools

import jax
import jax.numpy as jnp
from jax import lax
from jax.experimental import pallas as pl
from jax.experimental.pallas import tpu as pltpu
from jax.experimental.pallas import tpu_sc as plsc

NUM_CORES = 2
NUM_SUBCORES = 16
LANES = 16
NUM_WORKERS = NUM_CORES * NUM_SUBCORES

POS_OFFSET = 2
CHUNK = 8
NHALF = 2


def _make_sc_kernel(B, S, H, dtype):
    assert S % NUM_WORKERS == 0
    s_per_w = S // NUM_WORKERS
    assert s_per_w % CHUNK == 0
    n_chunks = s_per_w // CHUNK
    HH = H // NHALF
    n_items = n_chunks * NHALF
    ROWS = B * CHUNK
    assert s_per_w % LANES == 0

    mesh = plsc.VectorSubcoreMesh(
        core_axis_name="c", subcore_axis_name="s",
        num_cores=NUM_CORES, num_subcores=NUM_SUBCORES)

    @functools.partial(
        pl.kernel,
        out_type=jax.ShapeDtypeStruct((B * S, H), dtype),
        mesh=mesh,
        scratch_types=[
            pltpu.VMEM((n_chunks, ROWS), jnp.int32),
            pltpu.VMEM((n_chunks, ROWS), jnp.int32),
            pltpu.VMEM((s_per_w,), jnp.int32),
            [pltpu.VMEM((ROWS, HH), dtype) for _ in range(2)],
            [pltpu.VMEM((CHUNK, H), dtype) for _ in range(2)],
            pltpu.SemaphoreType.DMA,
            pltpu.SemaphoreType.DMA,
            pltpu.SemaphoreType.DMA,
        ],
    )
    def body(ids_hbm, orow_hbm, tok_tbl_hbm, pos_tbl_hbm, out_hbm,
             idx_v, orow_v, pos_idx_v, tok_bufs, pos_vs, gsem, wsem, psem):
        cid = lax.axis_index("c")
        sid = lax.axis_index("s")
        wid = sid * NUM_CORES + cid
        s0 = wid * s_per_w
        c0 = wid * n_chunks

        pltpu.sync_copy(ids_hbm.at[pl.ds(c0, n_chunks)], idx_v)
        pltpu.sync_copy(orow_hbm.at[pl.ds(c0, n_chunks)], orow_v)
        for t in range(s_per_w // LANES):
            pos_idx_v[pl.ds(t * LANES, LANES)] = (
                lax.iota(jnp.int32, LANES) + (s0 + POS_OFFSET + t * LANES))

        def fire_gather(i):
            c, h = i // NHALF, i % NHALF
            return pltpu.async_copy(
                tok_tbl_hbm.at[idx_v.at[c], pl.ds(h * HH, HH)],
                tok_bufs[i % 2], gsem)

        def fire_pos(c):
            return pltpu.async_copy(
                pos_tbl_hbm.at[pos_idx_v.at[pl.ds(c * CHUNK, CHUNK)]],
                pos_vs[c % 2], psem)

        pend_g = {0: fire_gather(0)}
        pend_p = {0: fire_pos(0)}
        pend_w = {}

        for i in range(n_items):
            c, h = i // NHALF, i % NHALF
            if i + 1 < n_items:
                if i - 1 >= 0:
                    pend_w[i - 1].wait()
                pend_g[i + 1] = fire_gather(i + 1)
                if h == 0 and c + 1 < n_chunks:
                    pend_p[c + 1] = fire_pos(c + 1)

            pend_g[i].wait()
            if h == 0:
                pend_p[c].wait()

            buf = tok_bufs[i % 2]
            pos_v = pos_vs[c % 2]

            def add_vec(j, carry, _buf=buf, _pos=pos_v, _h=h):
                for r in range(CHUNK):
                    sl = pl.ds(j * LANES, LANES)
                    pv = _pos[r, pl.ds(_h * HH + j * LANES, LANES)]
                    for b in range(B):
                        _buf[b * CHUNK + r, sl] = _buf[b * CHUNK + r, sl] + pv
                return carry

            lax.fori_loop(0, HH // LANES, add_vec, 0)

            pend_w[i] = pltpu.async_copy(
                buf, out_hbm.at[orow_v.at[c], pl.ds(h * HH, HH)], wsem)

        pend_w[n_items - 2].wait()
        pend_w[n_items - 1].wait()

    return body


def kernel(token_ids, token_table, pos_table):
    B, S = token_ids.shape
    V, H = token_table.shape
    n_chunks_tot = S // CHUNK
    ids_r = (token_ids.astype(jnp.int32)
             .reshape(B, n_chunks_tot, CHUNK)
             .transpose(1, 0, 2)
             .reshape(n_chunks_tot, B * CHUNK))
    orow = (jnp.arange(B * S, dtype=jnp.int32)
            .reshape(B, n_chunks_tot, CHUNK)
            .transpose(1, 0, 2)
            .reshape(n_chunks_tot, B * CHUNK))
    sc = _make_sc_kernel(B, S, H, token_table.dtype)
    out = sc(ids_r, orow, token_table, pos_table)
    return out.reshape(B, S, H)


# speedup vs baseline: 1.0048x; 1.0048x over previous
import functools

import jax
import jax.numpy as jnp
from jax import lax
from jax.experimental import pallas as pl
from jax.experimental.pallas import tpu as pltpu
from jax.experimental.pallas import tpu_sc as plsc

NUM_CORES = 2
NUM_SUBCORES = 16
LANES = 16
NUM_WORKERS = NUM_CORES * NUM_SUBCORES

POS_OFFSET = 2
CHUNK = 8
NHALF = 4
NBUF = 4


def _make_sc_kernel(B, S, H, dtype):
    assert S % NUM_WORKERS == 0
    s_per_w = S // NUM_WORKERS
    assert s_per_w % CHUNK == 0
    n_chunks = s_per_w // CHUNK
    HH = H // NHALF
    n_items = n_chunks * NHALF
    ROWS = B * CHUNK
    assert s_per_w % LANES == 0

    mesh = plsc.VectorSubcoreMesh(
        core_axis_name="c", subcore_axis_name="s",
        num_cores=NUM_CORES, num_subcores=NUM_SUBCORES)

    @functools.partial(
        pl.kernel,
        out_type=jax.ShapeDtypeStruct((B * S, H), dtype),
        mesh=mesh,
        scratch_types=[
            pltpu.VMEM((n_chunks, ROWS), jnp.int32),
            pltpu.VMEM((n_chunks, ROWS), jnp.int32),
            pltpu.VMEM((s_per_w,), jnp.int32),
            [pltpu.VMEM((ROWS, HH), dtype) for _ in range(NBUF)],
            [pltpu.VMEM((CHUNK, H), dtype) for _ in range(2)],
            pltpu.SemaphoreType.DMA,
            pltpu.SemaphoreType.DMA,
            pltpu.SemaphoreType.DMA,
        ],
    )
    def body(ids_hbm, orow_hbm, tok_tbl_hbm, pos_tbl_hbm, out_hbm,
             idx_v, orow_v, pos_idx_v, tok_bufs, pos_vs, gsem, wsem, psem):
        cid = lax.axis_index("c")
        sid = lax.axis_index("s")
        wid = sid * NUM_CORES + cid
        s0 = wid * s_per_w
        c0 = wid * n_chunks

        pltpu.sync_copy(ids_hbm.at[pl.ds(c0, n_chunks)], idx_v)
        pltpu.sync_copy(orow_hbm.at[pl.ds(c0, n_chunks)], orow_v)
        for t in range(s_per_w // LANES):
            pos_idx_v[pl.ds(t * LANES, LANES)] = (
                lax.iota(jnp.int32, LANES) + (s0 + POS_OFFSET + t * LANES))

        def fire_gather(i):
            c, h = i // NHALF, i % NHALF
            return pltpu.async_copy(
                tok_tbl_hbm.at[idx_v.at[c], pl.ds(h * HH, HH)],
                tok_bufs[i % NBUF], gsem)

        def fire_pos(c):
            return pltpu.async_copy(
                pos_tbl_hbm.at[pos_idx_v.at[pl.ds(c * CHUNK, CHUNK)]],
                pos_vs[c % 2], psem)

        pend_g = {0: fire_gather(0)}
        pend_p = {0: fire_pos(0)}
        pend_w = {}

        for i in range(n_items):
            c, h = i // NHALF, i % NHALF
            if i + 1 < n_items:
                if i + 1 - NBUF >= 0:
                    pend_w[i + 1 - NBUF].wait()
                pend_g[i + 1] = fire_gather(i + 1)
                if h == 0 and c + 1 < n_chunks:
                    pend_p[c + 1] = fire_pos(c + 1)

            pend_g[i].wait()
            if h == 0:
                pend_p[c].wait()

            buf = tok_bufs[i % NBUF]
            pos_v = pos_vs[c % 2]

            def add_vec(j, carry, _buf=buf, _pos=pos_v, _h=h):
                for r in range(CHUNK):
                    sl = pl.ds(j * LANES, LANES)
                    pv = _pos[r, pl.ds(_h * HH + j * LANES, LANES)]
                    for b in range(B):
                        _buf[b * CHUNK + r, sl] = _buf[b * CHUNK + r, sl] + pv
                return carry

            lax.fori_loop(0, HH // LANES, add_vec, 0)

            pend_w[i] = pltpu.async_copy(
                buf, out_hbm.at[orow_v.at[c], pl.ds(h * HH, HH)], wsem)

        for i in range(max(0, n_items - NBUF), n_items):
            pend_w[i].wait()

    return body


def kernel(token_ids, token_table, pos_table):
    B, S = token_ids.shape
    V, H = token_table.shape
    n_chunks_tot = S // CHUNK
    ids_r = (token_ids.astype(jnp.int32)
             .reshape(B, n_chunks_tot, CHUNK)
             .transpose(1, 0, 2)
             .reshape(n_chunks_tot, B * CHUNK))
    orow = (jnp.arange(B * S, dtype=jnp.int32)
            .reshape(B, n_chunks_tot, CHUNK)
            .transpose(1, 0, 2)
            .reshape(n_chunks_tot, B * CHUNK))
    sc = _make_sc_kernel(B, S, H, token_table.dtype)
    out = sc(ids_r, orow, token_table, pos_table)
    return out.reshape(B, S, H)
